# baseline (speedup 1.0000x reference)
import jax
import jax.numpy as jnp
from jax import lax
from jax.experimental import pallas as pl
from jax.experimental.pallas import tpu as pltpu
from jax.experimental.pallas import tpu_sc as plsc

B = 16384
EMB = 256
SRC = 256
NC = 2
NS = 16
NW = NC * NS
B_PER_W = B // NW
CH = 64
NCH = B_PER_W // CH
BM = 4096


def _sc_gather_body(table_hbm, idx_hbm, out_hbm, idx_v,
                    fb0, fb1, fb2, fb3, sg0, sg1, sg2, sg3,
                    ss0, ss1, ss2, ss3):
    wid = lax.axis_index("s") * NC + lax.axis_index("c")
    base = wid * B_PER_W
    pltpu.sync_copy(idx_hbm.at[pl.ds(base, B_PER_W)], idx_v)
    fbufs = (fb0, fb1, fb2, fb3)
    gsems = (sg0, sg1, sg2, sg3)
    ssems = (ss0, ss1, ss2, ss3)
    DEPTH = 3
    gd = [None] * NCH
    scat = [None, None, None, None]
    for c in range(min(DEPTH, NCH)):
        gd[c] = pltpu.async_copy(
            table_hbm.at[idx_v.at[pl.ds(c * CH, CH)]], fbufs[c % 4],
            gsems[c % 4])
    for c in range(NCH):
        gd[c].wait()
        nxt = c + DEPTH
        if nxt < NCH:
            nb = nxt % 4
            if scat[nb] is not None:
                scat[nb].wait()
            gd[nxt] = pltpu.async_copy(
                table_hbm.at[idx_v.at[pl.ds(nxt * CH, CH)]], fbufs[nb],
                gsems[nb])
        scat[c % 4] = pltpu.async_copy(
            fbufs[c % 4], out_hbm.at[pl.ds(base + c * CH, CH)], ssems[c % 4])
    for s in scat:
        if s is not None:
            s.wait()


_sc_gather = pl.kernel(
    _sc_gather_body,
    out_type=jax.ShapeDtypeStruct((B, EMB), jnp.float32),
    mesh=plsc.VectorSubcoreMesh(core_axis_name="c", subcore_axis_name="s"),
    compiler_params=pltpu.CompilerParams(needs_layout_passes=False),
    scratch_types=[
        pltpu.VMEM((B_PER_W,), jnp.int32),
        pltpu.VMEM((CH, EMB), jnp.float32),
        pltpu.VMEM((CH, EMB), jnp.float32),
        pltpu.VMEM((CH, EMB), jnp.float32),
        pltpu.VMEM((CH, EMB), jnp.float32),
        pltpu.SemaphoreType.DMA,
        pltpu.SemaphoreType.DMA,
        pltpu.SemaphoreType.DMA,
        pltpu.SemaphoreType.DMA,
        pltpu.SemaphoreType.DMA,
        pltpu.SemaphoreType.DMA,
        pltpu.SemaphoreType.DMA,
        pltpu.SemaphoreType.DMA,
    ],
)

_DN = (((1,), (1,)), ((), ()))


def _mm_body(x_ref, w0_ref, w1_ref, o_ref):
    x = x_ref[...]
    o_ref[:, 0, :] = lax.dot_general(x, w0_ref[...], _DN,
                                     preferred_element_type=jnp.float32)
    o_ref[:, 1, :] = lax.dot_general(x, w1_ref[...], _DN,
                                     preferred_element_type=jnp.float32)


_matmul = pl.pallas_call(
    _mm_body,
    grid=(B // BM,),
    in_specs=[
        pl.BlockSpec((BM, EMB), lambda i: (i, 0)),
        pl.BlockSpec((SRC, EMB), lambda i: (0, 0)),
        pl.BlockSpec((SRC, EMB), lambda i: (0, 0)),
    ],
    out_specs=pl.BlockSpec((BM, 2, SRC), lambda i: (i, 0, 0)),
    out_shape=jax.ShapeDtypeStruct((B, 2, SRC), jnp.float32),
)


@jax.jit
def _run(indexes, entity_table, W0, W1):
    emb = _sc_gather(entity_table, indexes)
    return _matmul(emb, W0, W1)


def kernel(indexes, entity_table, W0, W1):
    return _run(indexes, entity_table, W0, W1)
